# baseline (speedup 1.0000x reference)
import functools

import jax
import jax.numpy as jnp
from jax.experimental import pallas as pl
from jax.experimental.pallas import tpu as pltpu


def _encoder_kernel(x_ref, w1_ref, b1_ref, w2_ref, b2_ref, o_ref, xt_ref,
                    *, H, W, C):
    bf16 = jnp.bfloat16
    xt_ref[...] = jnp.transpose(x_ref[...])
    x2d = xt_ref[...]
    xs1 = jnp.concatenate([x2d[1:], x2d[:1]], axis=0)
    xs2 = jnp.concatenate([x2d[2:], x2d[:2]], axis=0)
    xp = jnp.concatenate(
        [x2d.astype(bf16), xs1.astype(bf16), xs2.astype(bf16)], axis=1)

    z1 = jnp.dot(xp, w1_ref[...], preferred_element_type=jnp.float32)

    M1 = (H - 2) * W
    h = (z1[0:M1, 0:C] + z1[W:M1 + W, C:2 * C]
         + z1[2 * W:M1 + 2 * W, 2 * C:3 * C] + b1_ref[...])
    h = jnp.maximum(h, 0.0)

    hs1 = jnp.concatenate([h[1:], h[:1]], axis=0)
    hs2 = jnp.concatenate([h[2:], h[:2]], axis=0)
    hp = jnp.concatenate(
        [h.astype(bf16), hs1.astype(bf16), hs2.astype(bf16)], axis=1)

    z2 = jnp.dot(hp, w2_ref[...], preferred_element_type=jnp.float32)

    M2 = (H - 4) * W
    y = (z2[0:M2, 0:C] + z2[W:M2 + W, C:2 * C]
         + z2[2 * W:M2 + 2 * W, 2 * C:3 * C] + b2_ref[...])
    y = jnp.maximum(y, 0.0)
    y = y + xs2[2 * W:2 * W + M2, :]

    yT = jnp.transpose(y)
    Wo = W - 4
    pieces = [yT[:, i * W:i * W + Wo] for i in range(H - 4)]
    o_ref[...] = jnp.concatenate(pieces, axis=1)


def kernel(x, w1, b1, w2, b2):
    N, C, H, W = x.shape
    bf16 = jnp.bfloat16
    xf = x.reshape(N, C, H * W)
    w1c = jnp.transpose(w1, (3, 1, 2, 0)).reshape(3 * C, 3 * C).astype(bf16)
    w2c = jnp.transpose(w2, (3, 1, 2, 0)).reshape(3 * C, 3 * C).astype(bf16)
    b1k = b1.reshape(1, C).astype(jnp.float32)
    b2k = b2.reshape(1, C).astype(jnp.float32)

    body = functools.partial(_encoder_kernel, H=H, W=W, C=C)
    out = pl.pallas_call(
        body,
        out_shape=jax.ShapeDtypeStruct((N, C, (H - 4) * (W - 4)), jnp.float32),
        grid=(N,),
        in_specs=[
            pl.BlockSpec((None, C, H * W), lambda b: (b, 0, 0)),
            pl.BlockSpec((3 * C, 3 * C), lambda b: (0, 0)),
            pl.BlockSpec((1, C), lambda b: (0, 0)),
            pl.BlockSpec((3 * C, 3 * C), lambda b: (0, 0)),
            pl.BlockSpec((1, C), lambda b: (0, 0)),
        ],
        out_specs=pl.BlockSpec((None, C, (H - 4) * (W - 4)),
                               lambda b: (b, 0, 0)),
        scratch_shapes=[pltpu.VMEM((H * W, C), jnp.float32)],
        compiler_params=pltpu.CompilerParams(
            dimension_semantics=("parallel",),
            vmem_limit_bytes=64 * 1024 * 1024),
    )(xf, w1c, b1k, w2c, b2k)
    return out.reshape(N, C, H - 4, W - 4)
